# baseline (speedup 1.0000x reference)
import functools

import jax
import jax.numpy as jnp
from jax import lax
from jax.experimental import pallas as pl
from jax.experimental.pallas import tpu as pltpu
from jax.experimental.pallas import tpu_sc as plsc

_INP = 416.0
_OFF = 418.0
_THRES = 0.3
_K = 8
_L = 16
_NPAD = 5120
_NSL = _NPAD // _L


def _nms_body(cx_h, cy_h, w_h, h_h, sc_h, cl_h, out_h,
              cx_v, cy_v, w_v, h_v, sc_v, cl_v,
              x1_v, y1_v, x2_v, y2_v, ar_v, out_v):
    @pl.when((lax.axis_index("c") == 0) & (lax.axis_index("s") == 0))
    def _():
        pltpu.sync_copy(cx_h, cx_v)
        pltpu.sync_copy(cy_h, cy_v)
        pltpu.sync_copy(w_h, w_v)
        pltpu.sync_copy(h_h, h_v)
        pltpu.sync_copy(sc_h, sc_v)
        pltpu.sync_copy(cl_h, cl_v)

        lane = lax.iota(jnp.int32, _L)
        neg2 = jnp.full((_L,), -2.0, jnp.float32)
        zero_i = jnp.zeros((_L,), jnp.int32)

        def conv_amax(i, carry):
            bv, bi = carry
            for u in range(4):
                j = 4 * i + u
                sl = pl.ds(j * _L, _L)
                cx = cx_v[sl]
                cy = cy_v[sl]
                w = w_v[sl]
                h = h_v[sl]
                off = cl_v[sl].astype(jnp.float32) * _OFF
                x1 = jnp.minimum(jnp.maximum(cx - w / 2.0, 0.0), _INP) + off
                y1 = jnp.minimum(jnp.maximum(cy - h / 2.0, 0.0), _INP) + off
                x2 = jnp.minimum(jnp.maximum(cx + w / 2.0, 0.0), _INP) + off
                y2 = jnp.minimum(jnp.maximum(cy + h / 2.0, 0.0), _INP) + off
                x1_v[sl] = x1
                y1_v[sl] = y1
                x2_v[sl] = x2
                y2_v[sl] = y2
                ar_v[sl] = (x2 - x1 + 1.0) * (y2 - y1 + 1.0)
                s = sc_v[sl]
                upd = s > bv
                bv = jnp.where(upd, s, bv)
                bi = jnp.where(upd, lane + j * _L, bi)
            return bv, bi

        bv, bi = lax.fori_loop(0, _NSL // 4, conv_amax, (neg2, zero_i))

        for k in range(_K):
            for s in (1, 2, 4, 8):
                p = lane ^ s
                bv2 = bv.at[p].get(mode="promise_in_bounds")
                bi2 = bi.at[p].get(mode="promise_in_bounds")
                take = (bv2 > bv) | ((bv2 == bv) & (bi2 < bi))
                bv = jnp.where(take, bv2, bv)
                bi = jnp.where(take, bi2, bi)
            mv = bv
            giv = bi

            px1 = plsc.load_gather(x1_v, [giv])
            py1 = plsc.load_gather(y1_v, [giv])
            px2 = plsc.load_gather(x2_v, [giv])
            py2 = plsc.load_gather(y2_v, [giv])
            pa = plsc.load_gather(ar_v, [giv])

            pcx = plsc.load_gather(cx_v, [giv])
            pcy = plsc.load_gather(cy_v, [giv])
            pw = plsc.load_gather(w_v, [giv])
            ph = plsc.load_gather(h_v, [giv])
            pcl = plsc.load_gather(cl_v, [giv]).astype(jnp.float32)
            ux1 = jnp.minimum(jnp.maximum(pcx - pw / 2.0, 0.0), _INP)
            uy1 = jnp.minimum(jnp.maximum(pcy - ph / 2.0, 0.0), _INP)
            ux2 = jnp.minimum(jnp.maximum(pcx + pw / 2.0, 0.0), _INP)
            uy2 = jnp.minimum(jnp.maximum(pcy + ph / 2.0, 0.0), _INP)
            row = jnp.where(lane == 0, ux1, 0.0)
            row = jnp.where(lane == 1, uy1, row)
            row = jnp.where(lane == 2, ux2, row)
            row = jnp.where(lane == 3, uy2, row)
            row = jnp.where(lane == 4, mv, row)
            row = jnp.where(lane == 5, pcl, row)
            out_v[pl.ds(k * _L, _L)] = row

            if k == _K - 1:
                break

            def sup_amax(i, carry):
                bv, bi = carry
                for u in range(4):
                    j = 4 * i + u
                    sl = pl.ds(j * _L, _L)
                    x1 = x1_v[sl]
                    y1 = y1_v[sl]
                    x2 = x2_v[sl]
                    y2 = y2_v[sl]
                    a = ar_v[sl]
                    ix1 = jnp.maximum(px1, x1)
                    iy1 = jnp.maximum(py1, y1)
                    ix2 = jnp.minimum(px2, x2)
                    iy2 = jnp.minimum(py2, y2)
                    inter = (jnp.maximum(ix2 - ix1 + 1.0, 0.0)
                             * jnp.maximum(iy2 - iy1 + 1.0, 0.0))
                    iou = inter / (pa + a - inter + 1e-16)
                    s = jnp.where(iou > _THRES, -1.0, sc_v[sl])
                    sc_v[sl] = s
                    upd = s > bv
                    bv = jnp.where(upd, s, bv)
                    bi = jnp.where(upd, lane + j * _L, bi)
                return bv, bi

            bv, bi = lax.fori_loop(0, _NSL // 4, sup_amax, (neg2, zero_i))

        pltpu.sync_copy(out_v, out_h)


_nms_sc = functools.partial(
    pl.kernel,
    out_type=jax.ShapeDtypeStruct((_K * _L,), jnp.float32),
    mesh=plsc.VectorSubcoreMesh(core_axis_name="c", subcore_axis_name="s"),
    compiler_params=pltpu.CompilerParams(needs_layout_passes=False),
    scratch_types=[
        pltpu.VMEM((_NPAD,), jnp.float32),
        pltpu.VMEM((_NPAD,), jnp.float32),
        pltpu.VMEM((_NPAD,), jnp.float32),
        pltpu.VMEM((_NPAD,), jnp.float32),
        pltpu.VMEM((_NPAD,), jnp.float32),
        pltpu.VMEM((_NPAD,), jnp.int32),
        pltpu.VMEM((_NPAD,), jnp.float32),
        pltpu.VMEM((_NPAD,), jnp.float32),
        pltpu.VMEM((_NPAD,), jnp.float32),
        pltpu.VMEM((_NPAD,), jnp.float32),
        pltpu.VMEM((_NPAD,), jnp.float32),
        pltpu.VMEM((_K * _L,), jnp.float32),
    ],
)(_nms_body)


def kernel(boxes, scores, idxs):
    n = boxes.shape[0]
    bp = jnp.zeros((_NPAD, 4), jnp.float32).at[:n].set(boxes)
    sp = jnp.full((_NPAD,), -1.0, jnp.float32).at[:n].set(scores)
    cp = jnp.zeros((_NPAD,), jnp.int32).at[:n].set(idxs)
    out = _nms_sc(bp[:, 0], bp[:, 1], bp[:, 2], bp[:, 3], sp, cp)
    return out.reshape(_K, _L)[:, :6]


# speedup vs baseline: 1.1876x; 1.1876x over previous
import functools

import jax
import jax.numpy as jnp
from jax import lax
from jax.experimental import pallas as pl
from jax.experimental.pallas import tpu as pltpu
from jax.experimental.pallas import tpu_sc as plsc

_INP = 416.0
_OFF = 418.0
_THRES = 0.3
_K = 8
_L = 16
_NPAD = 5120
_NW = 16
_C = _NPAD // _NW
_CI = _C // (2 * _L)


def _clip(v):
    return jnp.minimum(jnp.maximum(v, 0.0), _INP)


def _nms_body(cx_h, cy_h, w_h, h_h, sc_h, cl_h, out_h,
              cx_v, cy_v, w_v, h_v, cl_v,
              sc_c, x1_c, y1_c, x2_c, y2_c, ar_c,
              bvst_v, bist_v, bvall_v, biall_v, out_v,
              shv0, shi0, shv1, shi1):
    wid = lax.axis_index("s")
    base = wid * _C

    pltpu.sync_copy(cx_h, cx_v)
    pltpu.sync_copy(cy_h, cy_v)
    pltpu.sync_copy(w_h, w_v)
    pltpu.sync_copy(h_h, h_v)
    pltpu.sync_copy(cl_h, cl_v)
    pltpu.sync_copy(sc_h.at[pl.ds(base, _C)], sc_c)

    lane = lax.iota(jnp.int32, _L)
    neg2 = jnp.full((_L,), -2.0, jnp.float32)
    zero_i = jnp.zeros((_L,), jnp.int32)

    def conv_amax(i, carry):
        bv, bi = carry
        for u in range(2):
            j = 2 * i + u
            sl = pl.ds(j * _L, _L)
            gsl = pl.ds(base + j * _L, _L)
            cx = cx_v[gsl]
            cy = cy_v[gsl]
            w = w_v[gsl]
            h = h_v[gsl]
            off = cl_v[gsl].astype(jnp.float32) * _OFF
            x1 = _clip(cx - w / 2.0) + off
            y1 = _clip(cy - h / 2.0) + off
            x2 = _clip(cx + w / 2.0) + off
            y2 = _clip(cy + h / 2.0) + off
            x1_c[sl] = x1
            y1_c[sl] = y1
            x2_c[sl] = x2
            y2_c[sl] = y2
            ar_c[sl] = (x2 - x1 + 1.0) * (y2 - y1 + 1.0)
            s = sc_c[sl]
            upd = s > bv
            bv = jnp.where(upd, s, bv)
            bi = jnp.where(upd, lane + (base + j * _L), bi)
        return bv, bi

    bv, bi = lax.fori_loop(0, _CI, conv_amax, (neg2, zero_i))

    for k in range(_K):
        shv, shi = (shv0, shi0) if k % 2 == 0 else (shv1, shi1)
        bvst_v[...] = bv
        bist_v[...] = bi
        pltpu.sync_copy(bvst_v, shv.at[pl.ds(wid * _L, _L)])
        pltpu.sync_copy(bist_v, shi.at[pl.ds(wid * _L, _L)])
        plsc.subcore_barrier()
        pltpu.sync_copy(shv, bvall_v)
        pltpu.sync_copy(shi, biall_v)

        bv = bvall_v[pl.ds(0, _L)]
        bi = biall_v[pl.ds(0, _L)]
        for r in range(1, _NW):
            v2 = bvall_v[pl.ds(r * _L, _L)]
            i2 = biall_v[pl.ds(r * _L, _L)]
            take = (v2 > bv) | ((v2 == bv) & (i2 < bi))
            bv = jnp.where(take, v2, bv)
            bi = jnp.where(take, i2, bi)
        for s in (1, 2, 4, 8):
            p = lane ^ s
            bv2 = bv.at[p].get(mode="promise_in_bounds")
            bi2 = bi.at[p].get(mode="promise_in_bounds")
            take = (bv2 > bv) | ((bv2 == bv) & (bi2 < bi))
            bv = jnp.where(take, bv2, bv)
            bi = jnp.where(take, bi2, bi)
        mv = bv
        giv = bi

        pcx = plsc.load_gather(cx_v, [giv])
        pcy = plsc.load_gather(cy_v, [giv])
        pw = plsc.load_gather(w_v, [giv])
        ph = plsc.load_gather(h_v, [giv])
        pcl = plsc.load_gather(cl_v, [giv]).astype(jnp.float32)
        offp = pcl * _OFF
        ux1 = _clip(pcx - pw / 2.0)
        uy1 = _clip(pcy - ph / 2.0)
        ux2 = _clip(pcx + pw / 2.0)
        uy2 = _clip(pcy + ph / 2.0)
        px1 = ux1 + offp
        py1 = uy1 + offp
        px2 = ux2 + offp
        py2 = uy2 + offp
        pa = (px2 - px1 + 1.0) * (py2 - py1 + 1.0)

        row = jnp.where(lane == 0, ux1, 0.0)
        row = jnp.where(lane == 1, uy1, row)
        row = jnp.where(lane == 2, ux2, row)
        row = jnp.where(lane == 3, uy2, row)
        row = jnp.where(lane == 4, mv, row)
        row = jnp.where(lane == 5, pcl, row)
        out_v[pl.ds(k * _L, _L)] = row

        if k == _K - 1:
            break

        def sup_amax(i, carry):
            bv, bi = carry
            for u in range(2):
                j = 2 * i + u
                sl = pl.ds(j * _L, _L)
                x1 = x1_c[sl]
                y1 = y1_c[sl]
                x2 = x2_c[sl]
                y2 = y2_c[sl]
                a = ar_c[sl]
                ix1 = jnp.maximum(px1, x1)
                iy1 = jnp.maximum(py1, y1)
                ix2 = jnp.minimum(px2, x2)
                iy2 = jnp.minimum(py2, y2)
                inter = (jnp.maximum(ix2 - ix1 + 1.0, 0.0)
                         * jnp.maximum(iy2 - iy1 + 1.0, 0.0))
                iou = inter / (pa + a - inter + 1e-16)
                s = jnp.where(iou > _THRES, -1.0, sc_c[sl])
                sc_c[sl] = s
                upd = s > bv
                bv = jnp.where(upd, s, bv)
                bi = jnp.where(upd, lane + (base + j * _L), bi)
            return bv, bi

        bv, bi = lax.fori_loop(0, _CI, sup_amax, (neg2, zero_i))

    @pl.when(wid == 0)
    def _():
        pltpu.sync_copy(out_v, out_h)


_nms_sc = functools.partial(
    pl.kernel,
    out_type=jax.ShapeDtypeStruct((_K * _L,), jnp.float32),
    mesh=plsc.VectorSubcoreMesh(
        core_axis_name="c", subcore_axis_name="s", num_cores=1),
    compiler_params=pltpu.CompilerParams(needs_layout_passes=False),
    scratch_types=[
        pltpu.VMEM((_NPAD,), jnp.float32),
        pltpu.VMEM((_NPAD,), jnp.float32),
        pltpu.VMEM((_NPAD,), jnp.float32),
        pltpu.VMEM((_NPAD,), jnp.float32),
        pltpu.VMEM((_NPAD,), jnp.int32),
        pltpu.VMEM((_C,), jnp.float32),
        pltpu.VMEM((_C,), jnp.float32),
        pltpu.VMEM((_C,), jnp.float32),
        pltpu.VMEM((_C,), jnp.float32),
        pltpu.VMEM((_C,), jnp.float32),
        pltpu.VMEM((_C,), jnp.float32),
        pltpu.VMEM((_L,), jnp.float32),
        pltpu.VMEM((_L,), jnp.int32),
        pltpu.VMEM((_NW * _L,), jnp.float32),
        pltpu.VMEM((_NW * _L,), jnp.int32),
        pltpu.VMEM((_K * _L,), jnp.float32),
        pltpu.VMEM_SHARED((_NW * _L,), jnp.float32),
        pltpu.VMEM_SHARED((_NW * _L,), jnp.int32),
        pltpu.VMEM_SHARED((_NW * _L,), jnp.float32),
        pltpu.VMEM_SHARED((_NW * _L,), jnp.int32),
    ],
)(_nms_body)


def kernel(boxes, scores, idxs):
    n = boxes.shape[0]
    bp = jnp.zeros((_NPAD, 4), jnp.float32).at[:n].set(boxes)
    sp = jnp.full((_NPAD,), -1.0, jnp.float32).at[:n].set(scores)
    cp = jnp.zeros((_NPAD,), jnp.int32).at[:n].set(idxs)
    out = _nms_sc(bp[:, 0], bp[:, 1], bp[:, 2], bp[:, 3], sp, cp)
    return out.reshape(_K, _L)[:, :6]


# speedup vs baseline: 1.3606x; 1.1457x over previous
import functools

import jax
import jax.numpy as jnp
from jax import lax
from jax.experimental import pallas as pl
from jax.experimental.pallas import tpu as pltpu
from jax.experimental.pallas import tpu_sc as plsc

_INP = 416.0
_OFF = 418.0
_THRES = 0.3
_K = 8
_L = 16
_NPAD = 5120
_NW = 16
_C = _NPAD // _NW
_CI = _C // (2 * _L)


def _clip(v):
    return jnp.minimum(jnp.maximum(v, 0.0), _INP)


def _nms_body(cx_h, cy_h, w_h, h_h, sc_h, cl_h, out_h,
              cx_v, cy_v, w_v, h_v, cl_v,
              sc_c, x1_c, y1_c, x2_c, y2_c, ar_c,
              st_v, all_v, out_v,
              sh0, sh1, dsem):
    wid = lax.axis_index("s")
    base = wid * _C

    cps = [
        pltpu.async_copy(cx_h, cx_v, dsem),
        pltpu.async_copy(cy_h, cy_v, dsem),
        pltpu.async_copy(w_h, w_v, dsem),
        pltpu.async_copy(h_h, h_v, dsem),
        pltpu.async_copy(cl_h, cl_v, dsem),
        pltpu.async_copy(sc_h.at[pl.ds(base, _C)], sc_c, dsem),
    ]
    for cp in cps:
        cp.wait()

    lane = lax.iota(jnp.int32, _L)
    neg2 = jnp.full((_L,), -2.0, jnp.float32)
    zero_i = jnp.zeros((_L,), jnp.int32)

    def conv_amax(i, carry):
        bv, bi = carry
        for u in range(2):
            j = 2 * i + u
            sl = pl.ds(j * _L, _L)
            gsl = pl.ds(base + j * _L, _L)
            cx = cx_v[gsl]
            cy = cy_v[gsl]
            w = w_v[gsl]
            h = h_v[gsl]
            off = cl_v[gsl].astype(jnp.float32) * _OFF
            x1 = _clip(cx - w / 2.0) + off
            y1 = _clip(cy - h / 2.0) + off
            x2 = _clip(cx + w / 2.0) + off
            y2 = _clip(cy + h / 2.0) + off
            x1_c[sl] = x1
            y1_c[sl] = y1
            x2_c[sl] = x2
            y2_c[sl] = y2
            ar_c[sl] = (x2 - x1 + 1.0) * (y2 - y1 + 1.0)
            s = sc_c[sl]
            upd = s > bv
            bv = jnp.where(upd, s, bv)
            bi = jnp.where(upd, lane + (base + j * _L), bi)
        return bv, bi

    bv, bi = lax.fori_loop(0, _CI, conv_amax, (neg2, zero_i))

    for k in range(_K):
        sh = sh0 if k % 2 == 0 else sh1
        st_v[pl.ds(0, _L)] = bv
        st_v[pl.ds(_L, _L)] = plsc.bitcast(bi, jnp.float32)
        pltpu.sync_copy(st_v, sh.at[pl.ds(wid * 2 * _L, 2 * _L)])
        plsc.subcore_barrier()
        pltpu.sync_copy(sh, all_v)

        bv = all_v[pl.ds(0, _L)]
        bi = plsc.bitcast(all_v[pl.ds(_L, _L)], jnp.int32)
        for r in range(1, _NW):
            v2 = all_v[pl.ds(r * 2 * _L, _L)]
            i2 = plsc.bitcast(all_v[pl.ds(r * 2 * _L + _L, _L)], jnp.int32)
            take = (v2 > bv) | ((v2 == bv) & (i2 < bi))
            bv = jnp.where(take, v2, bv)
            bi = jnp.where(take, i2, bi)
        for s in (1, 2, 4, 8):
            p = lane ^ s
            bv2 = bv.at[p].get(mode="promise_in_bounds")
            bi2 = bi.at[p].get(mode="promise_in_bounds")
            take = (bv2 > bv) | ((bv2 == bv) & (bi2 < bi))
            bv = jnp.where(take, bv2, bv)
            bi = jnp.where(take, bi2, bi)
        mv = bv
        giv = bi

        pcx = plsc.load_gather(cx_v, [giv])
        pcy = plsc.load_gather(cy_v, [giv])
        pw = plsc.load_gather(w_v, [giv])
        ph = plsc.load_gather(h_v, [giv])
        pcl = plsc.load_gather(cl_v, [giv]).astype(jnp.float32)
        offp = pcl * _OFF
        ux1 = _clip(pcx - pw / 2.0)
        uy1 = _clip(pcy - ph / 2.0)
        ux2 = _clip(pcx + pw / 2.0)
        uy2 = _clip(pcy + ph / 2.0)
        px1 = ux1 + offp
        py1 = uy1 + offp
        px2 = ux2 + offp
        py2 = uy2 + offp
        pa = (px2 - px1 + 1.0) * (py2 - py1 + 1.0)

        row = jnp.where(lane == 0, ux1, 0.0)
        row = jnp.where(lane == 1, uy1, row)
        row = jnp.where(lane == 2, ux2, row)
        row = jnp.where(lane == 3, uy2, row)
        row = jnp.where(lane == 4, mv, row)
        row = jnp.where(lane == 5, pcl, row)
        out_v[pl.ds(k * _L, _L)] = row

        if k == _K - 1:
            break

        def sup_amax(i, carry):
            bv, bi = carry
            for u in range(2):
                j = 2 * i + u
                sl = pl.ds(j * _L, _L)
                x1 = x1_c[sl]
                y1 = y1_c[sl]
                x2 = x2_c[sl]
                y2 = y2_c[sl]
                a = ar_c[sl]
                ix1 = jnp.maximum(px1, x1)
                iy1 = jnp.maximum(py1, y1)
                ix2 = jnp.minimum(px2, x2)
                iy2 = jnp.minimum(py2, y2)
                inter = (jnp.maximum(ix2 - ix1 + 1.0, 0.0)
                         * jnp.maximum(iy2 - iy1 + 1.0, 0.0))
                iou = inter / (pa + a - inter + 1e-16)
                s = jnp.where(iou > _THRES, -1.0, sc_c[sl])
                sc_c[sl] = s
                upd = s > bv
                bv = jnp.where(upd, s, bv)
                bi = jnp.where(upd, lane + (base + j * _L), bi)
            return bv, bi

        bv, bi = lax.fori_loop(0, _CI, sup_amax, (neg2, zero_i))

    @pl.when(wid == 0)
    def _():
        pltpu.sync_copy(out_v, out_h)


_nms_sc = functools.partial(
    pl.kernel,
    out_type=jax.ShapeDtypeStruct((_K * _L,), jnp.float32),
    mesh=plsc.VectorSubcoreMesh(
        core_axis_name="c", subcore_axis_name="s", num_cores=1),
    compiler_params=pltpu.CompilerParams(needs_layout_passes=False),
    scratch_types=[
        pltpu.VMEM((_NPAD,), jnp.float32),
        pltpu.VMEM((_NPAD,), jnp.float32),
        pltpu.VMEM((_NPAD,), jnp.float32),
        pltpu.VMEM((_NPAD,), jnp.float32),
        pltpu.VMEM((_NPAD,), jnp.int32),
        pltpu.VMEM((_C,), jnp.float32),
        pltpu.VMEM((_C,), jnp.float32),
        pltpu.VMEM((_C,), jnp.float32),
        pltpu.VMEM((_C,), jnp.float32),
        pltpu.VMEM((_C,), jnp.float32),
        pltpu.VMEM((_C,), jnp.float32),
        pltpu.VMEM((2 * _L,), jnp.float32),
        pltpu.VMEM((_NW * 2 * _L,), jnp.float32),
        pltpu.VMEM((_K * _L,), jnp.float32),
        pltpu.VMEM_SHARED((_NW * 2 * _L,), jnp.float32),
        pltpu.VMEM_SHARED((_NW * 2 * _L,), jnp.float32),
        pltpu.SemaphoreType.DMA,
    ],
)(_nms_body)


def kernel(boxes, scores, idxs):
    n = boxes.shape[0]
    bp = jnp.zeros((_NPAD, 4), jnp.float32).at[:n].set(boxes)
    sp = jnp.full((_NPAD,), -1.0, jnp.float32).at[:n].set(scores)
    cp = jnp.zeros((_NPAD,), jnp.int32).at[:n].set(idxs)
    out = _nms_sc(bp[:, 0], bp[:, 1], bp[:, 2], bp[:, 3], sp, cp)
    return out.reshape(_K, _L)[:, :6]


# speedup vs baseline: 1.3747x; 1.0104x over previous
import functools

import jax
import jax.numpy as jnp
from jax import lax
from jax.experimental import pallas as pl
from jax.experimental.pallas import tpu as pltpu
from jax.experimental.pallas import tpu_sc as plsc

_INP = 416.0
_OFF = 418.0
_THRES = 0.3
_K = 8
_L = 16
_N = 5000
_NPAD = 5120
_NW = 16
_C = _NPAD // _NW
_CI = _C // (2 * _L)
_TAIL = _N - (_NW - 1) * _C


def _clip(v):
    return jnp.minimum(jnp.maximum(v, 0.0), _INP)


def _nms_body(bx_h, sc_h, cl_h, out_h,
              bx_v, cl_v,
              sc_c, x1_c, y1_c, x2_c, y2_c, ar_c,
              st_v, all_v, out_v,
              sh0, sh1, dsem):
    wid = lax.axis_index("s")
    base = wid * _C

    cps = [
        pltpu.async_copy(bx_h, bx_v, dsem),
        pltpu.async_copy(cl_h, cl_v, dsem),
    ]
    @pl.when(wid < _NW - 1)
    def _():
        pltpu.async_copy(sc_h.at[pl.ds(base, _C)], sc_c, dsem).wait()
    @pl.when(wid == _NW - 1)
    def _():
        pltpu.async_copy(
            sc_h.at[pl.ds((_NW - 1) * _C, _TAIL)],
            sc_c.at[pl.ds(0, _TAIL)], dsem).wait()
    for cp in cps:
        cp.wait()

    lane = lax.iota(jnp.int32, _L)
    neg2 = jnp.full((_L,), -2.0, jnp.float32)
    zero_i = jnp.zeros((_L,), jnp.int32)

    def conv_amax(i, carry):
        bv, bi = carry
        for u in range(2):
            j = 2 * i + u
            sl = pl.ds(j * _L, _L)
            gidx = lane + (base + j * _L)
            g4 = jnp.minimum(gidx, _N - 1) * 4
            cx = plsc.load_gather(bx_v, [g4])
            cy = plsc.load_gather(bx_v, [g4 + 1])
            w = plsc.load_gather(bx_v, [g4 + 2])
            h = plsc.load_gather(bx_v, [g4 + 3])
            off = plsc.load_gather(
                cl_v, [jnp.minimum(gidx, _N - 1)]).astype(jnp.float32) * _OFF
            x1 = _clip(cx - w / 2.0) + off
            y1 = _clip(cy - h / 2.0) + off
            x2 = _clip(cx + w / 2.0) + off
            y2 = _clip(cy + h / 2.0) + off
            x1_c[sl] = x1
            y1_c[sl] = y1
            x2_c[sl] = x2
            y2_c[sl] = y2
            ar_c[sl] = (x2 - x1 + 1.0) * (y2 - y1 + 1.0)
            s = jnp.where(gidx < _N, sc_c[sl], -1.0)
            sc_c[sl] = s
            upd = s > bv
            bv = jnp.where(upd, s, bv)
            bi = jnp.where(upd, gidx, bi)
        return bv, bi

    bv, bi = lax.fori_loop(0, _CI, conv_amax, (neg2, zero_i))

    for k in range(_K):
        sh = sh0 if k % 2 == 0 else sh1
        st_v[pl.ds(0, _L)] = bv
        st_v[pl.ds(_L, _L)] = plsc.bitcast(bi, jnp.float32)
        pltpu.sync_copy(st_v, sh.at[pl.ds(wid * 2 * _L, 2 * _L)])
        plsc.subcore_barrier()
        pltpu.sync_copy(sh, all_v)

        bv = all_v[pl.ds(0, _L)]
        bi = plsc.bitcast(all_v[pl.ds(_L, _L)], jnp.int32)
        for r in range(1, _NW):
            v2 = all_v[pl.ds(r * 2 * _L, _L)]
            i2 = plsc.bitcast(all_v[pl.ds(r * 2 * _L + _L, _L)], jnp.int32)
            take = (v2 > bv) | ((v2 == bv) & (i2 < bi))
            bv = jnp.where(take, v2, bv)
            bi = jnp.where(take, i2, bi)
        for s in (1, 2, 4, 8):
            p = lane ^ s
            bv2 = bv.at[p].get(mode="promise_in_bounds")
            bi2 = bi.at[p].get(mode="promise_in_bounds")
            take = (bv2 > bv) | ((bv2 == bv) & (bi2 < bi))
            bv = jnp.where(take, bv2, bv)
            bi = jnp.where(take, bi2, bi)
        mv = bv
        giv = bi

        gp4 = jnp.minimum(giv, _N - 1) * 4
        pcx = plsc.load_gather(bx_v, [gp4])
        pcy = plsc.load_gather(bx_v, [gp4 + 1])
        pw = plsc.load_gather(bx_v, [gp4 + 2])
        ph = plsc.load_gather(bx_v, [gp4 + 3])
        pcl = plsc.load_gather(
            cl_v, [jnp.minimum(giv, _N - 1)]).astype(jnp.float32)
        offp = pcl * _OFF
        ux1 = _clip(pcx - pw / 2.0)
        uy1 = _clip(pcy - ph / 2.0)
        ux2 = _clip(pcx + pw / 2.0)
        uy2 = _clip(pcy + ph / 2.0)
        px1 = ux1 + offp
        py1 = uy1 + offp
        px2 = ux2 + offp
        py2 = uy2 + offp
        pa = (px2 - px1 + 1.0) * (py2 - py1 + 1.0)

        row = jnp.where(lane == 0, ux1, 0.0)
        row = jnp.where(lane == 1, uy1, row)
        row = jnp.where(lane == 2, ux2, row)
        row = jnp.where(lane == 3, uy2, row)
        row = jnp.where(lane == 4, mv, row)
        row = jnp.where(lane == 5, pcl, row)
        out_v[pl.ds(k * _L, _L)] = row

        if k == _K - 1:
            break

        def sup_amax(i, carry):
            bv, bi = carry
            for u in range(2):
                j = 2 * i + u
                sl = pl.ds(j * _L, _L)
                x1 = x1_c[sl]
                y1 = y1_c[sl]
                x2 = x2_c[sl]
                y2 = y2_c[sl]
                a = ar_c[sl]
                ix1 = jnp.maximum(px1, x1)
                iy1 = jnp.maximum(py1, y1)
                ix2 = jnp.minimum(px2, x2)
                iy2 = jnp.minimum(py2, y2)
                inter = (jnp.maximum(ix2 - ix1 + 1.0, 0.0)
                         * jnp.maximum(iy2 - iy1 + 1.0, 0.0))
                iou = inter / (pa + a - inter + 1e-16)
                s = jnp.where(iou > _THRES, -1.0, sc_c[sl])
                sc_c[sl] = s
                upd = s > bv
                bv = jnp.where(upd, s, bv)
                bi = jnp.where(upd, lane + (base + j * _L), bi)
            return bv, bi

        bv, bi = lax.fori_loop(0, _CI, sup_amax, (neg2, zero_i))

    @pl.when(wid == 0)
    def _():
        pltpu.sync_copy(out_v, out_h)


_nms_sc = functools.partial(
    pl.kernel,
    out_type=jax.ShapeDtypeStruct((_K * _L,), jnp.float32),
    mesh=plsc.VectorSubcoreMesh(
        core_axis_name="c", subcore_axis_name="s", num_cores=1),
    compiler_params=pltpu.CompilerParams(needs_layout_passes=False),
    scratch_types=[
        pltpu.VMEM((_N * 4,), jnp.float32),
        pltpu.VMEM((_N,), jnp.int32),
        pltpu.VMEM((_C,), jnp.float32),
        pltpu.VMEM((_C,), jnp.float32),
        pltpu.VMEM((_C,), jnp.float32),
        pltpu.VMEM((_C,), jnp.float32),
        pltpu.VMEM((_C,), jnp.float32),
        pltpu.VMEM((_C,), jnp.float32),
        pltpu.VMEM((2 * _L,), jnp.float32),
        pltpu.VMEM((_NW * 2 * _L,), jnp.float32),
        pltpu.VMEM((_K * _L,), jnp.float32),
        pltpu.VMEM_SHARED((_NW * 2 * _L,), jnp.float32),
        pltpu.VMEM_SHARED((_NW * 2 * _L,), jnp.float32),
        pltpu.SemaphoreType.DMA,
    ],
)(_nms_body)


def kernel(boxes, scores, idxs):
    out = _nms_sc(jnp.ravel(boxes), scores, idxs)
    return out.reshape(_K, _L)[:, :6]


# speedup vs baseline: 1.4379x; 1.0459x over previous
import functools

import jax
import jax.numpy as jnp
from jax import lax
from jax.experimental import pallas as pl
from jax.experimental.pallas import tpu as pltpu
from jax.experimental.pallas import tpu_sc as plsc

_INP = 416.0
_OFF = 418.0
_THRES = 0.3
_K = 8
_L = 16
_N = 5000
_NPAD = 5120
_NW = 16
_C = _NPAD // _NW
_CI = _C // (2 * _L)
_TAIL = _N - (_NW - 1) * _C


def _clip(v):
    return jnp.minimum(jnp.maximum(v, 0.0), _INP)


def _better(v2, i2, bv, bi):
    take = (v2 > bv) | ((v2 == bv) & (i2 < bi))
    return jnp.where(take, v2, bv), jnp.where(take, i2, bi)


def _nms_body(bx_h, sc_h, cl_h, out_h,
              bx_v, cl_v,
              sc_c, x1_c, y1_c, x2_c, y2_c, ar_c,
              st_v, all_v, out_v,
              sh0, sh1, dsem):
    wid = lax.axis_index("s")
    base = wid * _C

    cps = [
        pltpu.async_copy(bx_h, bx_v, dsem),
        pltpu.async_copy(cl_h, cl_v, dsem),
    ]
    @pl.when(wid < _NW - 1)
    def _():
        pltpu.async_copy(sc_h.at[pl.ds(base, _C)], sc_c, dsem).wait()
    @pl.when(wid == _NW - 1)
    def _():
        pltpu.async_copy(
            sc_h.at[pl.ds((_NW - 1) * _C, _TAIL)],
            sc_c.at[pl.ds(0, _TAIL)], dsem).wait()
    for cp in cps:
        cp.wait()

    lane = lax.iota(jnp.int32, _L)
    neg2 = jnp.full((_L,), -2.0, jnp.float32)
    zero_i = jnp.zeros((_L,), jnp.int32)

    def conv_amax(i, carry):
        bv, bi = carry
        for u in range(2):
            j = 2 * i + u
            sl = pl.ds(j * _L, _L)
            gidx = lane + (base + j * _L)
            gc = jnp.minimum(gidx, _N - 1)
            cx = plsc.load_gather(bx_v, [gc, zero_i])
            cy = plsc.load_gather(bx_v, [gc, zero_i + 1])
            w = plsc.load_gather(bx_v, [gc, zero_i + 2])
            h = plsc.load_gather(bx_v, [gc, zero_i + 3])
            off = plsc.load_gather(cl_v, [gc]).astype(jnp.float32) * _OFF
            x1 = _clip(cx - w / 2.0) + off
            y1 = _clip(cy - h / 2.0) + off
            x2 = _clip(cx + w / 2.0) + off
            y2 = _clip(cy + h / 2.0) + off
            x1_c[sl] = x1
            y1_c[sl] = y1
            x2_c[sl] = x2
            y2_c[sl] = y2
            ar_c[sl] = (x2 - x1 + 1.0) * (y2 - y1 + 1.0)
            s = jnp.where(gidx < _N, sc_c[sl], -1.0)
            sc_c[sl] = s
            upd = s > bv
            bv = jnp.where(upd, s, bv)
            bi = jnp.where(upd, gidx, bi)
        return bv, bi

    bv, bi = lax.fori_loop(0, _CI, conv_amax, (neg2, zero_i))

    def nms_round(k, carry):
        bv, bi = carry

        st_v[pl.ds(0, _L)] = bv
        st_v[pl.ds(_L, _L)] = plsc.bitcast(bi, jnp.float32)
        even = k % 2 == 0

        @pl.when(even)
        def _():
            pltpu.sync_copy(st_v, sh0.at[pl.ds(wid * 2 * _L, 2 * _L)])
        @pl.when(jnp.logical_not(even))
        def _():
            pltpu.sync_copy(st_v, sh1.at[pl.ds(wid * 2 * _L, 2 * _L)])
        plsc.subcore_barrier()
        @pl.when(even)
        def _():
            pltpu.sync_copy(sh0, all_v)
        @pl.when(jnp.logical_not(even))
        def _():
            pltpu.sync_copy(sh1, all_v)

        bv = all_v[pl.ds(0, _L)]
        bi = plsc.bitcast(all_v[pl.ds(_L, _L)], jnp.int32)
        for r in range(1, _NW):
            v2 = all_v[pl.ds(r * 2 * _L, _L)]
            i2 = plsc.bitcast(all_v[pl.ds(r * 2 * _L + _L, _L)], jnp.int32)
            bv, bi = _better(v2, i2, bv, bi)
        for s in (1, 2, 4, 8):
            p = lane ^ s
            bv2 = bv.at[p].get(mode="promise_in_bounds")
            bi2 = bi.at[p].get(mode="promise_in_bounds")
            bv, bi = _better(bv2, bi2, bv, bi)
        mv = bv
        giv = bi

        gp = jnp.minimum(giv, _N - 1)
        pcx = plsc.load_gather(bx_v, [gp, zero_i])
        pcy = plsc.load_gather(bx_v, [gp, zero_i + 1])
        pw = plsc.load_gather(bx_v, [gp, zero_i + 2])
        ph = plsc.load_gather(bx_v, [gp, zero_i + 3])
        pcl = plsc.load_gather(cl_v, [gp]).astype(jnp.float32)
        offp = pcl * _OFF
        ux1 = _clip(pcx - pw / 2.0)
        uy1 = _clip(pcy - ph / 2.0)
        ux2 = _clip(pcx + pw / 2.0)
        uy2 = _clip(pcy + ph / 2.0)
        px1 = ux1 + offp
        py1 = uy1 + offp
        px2 = ux2 + offp
        py2 = uy2 + offp
        pa = (px2 - px1 + 1.0) * (py2 - py1 + 1.0)

        row = jnp.where(lane == 0, ux1, 0.0)
        row = jnp.where(lane == 1, uy1, row)
        row = jnp.where(lane == 2, ux2, row)
        row = jnp.where(lane == 3, uy2, row)
        row = jnp.where(lane == 4, mv, row)
        row = jnp.where(lane == 5, pcl, row)
        out_v[pl.ds(k * _L, _L)] = row

        def sup_amax(i, carry):
            bv, bi = carry
            for u in range(2):
                j = 2 * i + u
                sl = pl.ds(j * _L, _L)
                x1 = x1_c[sl]
                y1 = y1_c[sl]
                x2 = x2_c[sl]
                y2 = y2_c[sl]
                a = ar_c[sl]
                ix1 = jnp.maximum(px1, x1)
                iy1 = jnp.maximum(py1, y1)
                ix2 = jnp.minimum(px2, x2)
                iy2 = jnp.minimum(py2, y2)
                inter = (jnp.maximum(ix2 - ix1 + 1.0, 0.0)
                         * jnp.maximum(iy2 - iy1 + 1.0, 0.0))
                iou = inter / (pa + a - inter + 1e-16)
                s = jnp.where(iou > _THRES, -1.0, sc_c[sl])
                sc_c[sl] = s
                upd = s > bv
                bv = jnp.where(upd, s, bv)
                bi = jnp.where(upd, lane + (base + j * _L), bi)
            return bv, bi

        return lax.fori_loop(0, _CI, sup_amax, (neg2, zero_i))

    lax.fori_loop(0, _K, nms_round, (bv, bi))

    @pl.when(wid == 0)
    def _():
        pltpu.sync_copy(out_v, out_h)


_nms_sc = functools.partial(
    pl.kernel,
    out_type=jax.ShapeDtypeStruct((_K * _L,), jnp.float32),
    mesh=plsc.VectorSubcoreMesh(
        core_axis_name="c", subcore_axis_name="s", num_cores=1),
    compiler_params=pltpu.CompilerParams(
        needs_layout_passes=False, use_tc_tiling_on_sc=False),
    scratch_types=[
        pltpu.VMEM((_N, 4), jnp.float32),
        pltpu.VMEM((_N,), jnp.int32),
        pltpu.VMEM((_C,), jnp.float32),
        pltpu.VMEM((_C,), jnp.float32),
        pltpu.VMEM((_C,), jnp.float32),
        pltpu.VMEM((_C,), jnp.float32),
        pltpu.VMEM((_C,), jnp.float32),
        pltpu.VMEM((_C,), jnp.float32),
        pltpu.VMEM((2 * _L,), jnp.float32),
        pltpu.VMEM((_NW * 2 * _L,), jnp.float32),
        pltpu.VMEM((_K * _L,), jnp.float32),
        pltpu.VMEM_SHARED((_NW * 2 * _L,), jnp.float32),
        pltpu.VMEM_SHARED((_NW * 2 * _L,), jnp.float32),
        pltpu.SemaphoreType.DMA,
    ],
)(_nms_body)


def kernel(boxes, scores, idxs):
    out = _nms_sc(boxes, scores, idxs)
    return out.reshape(_K, _L)[:, :6]


# speedup vs baseline: 1.5120x; 1.0515x over previous
import functools

import jax
import jax.numpy as jnp
from jax import lax
from jax.experimental import pallas as pl
from jax.experimental.pallas import tpu as pltpu
from jax.experimental.pallas import tpu_sc as plsc

_INP = 416.0
_OFF = 418.0
_THRES = 0.3
_K = 8
_L = 16
_N = 5000
_NPAD = 5120
_NW = 16
_C = _NPAD // _NW
_CI = _C // (2 * _L)
_TAIL = _N - (_NW - 1) * _C


def _clip(v):
    return jnp.minimum(jnp.maximum(v, 0.0), _INP)


def _better(v2, i2, bv, bi):
    take = (v2 > bv) | ((v2 == bv) & (i2 < bi))
    return jnp.where(take, v2, bv), jnp.where(take, i2, bi)


def _nms_body(cx_h, cy_h, w_h, h_h, sc_h, cl_h, out_h,
              cx_v, cy_v, w_v, h_v, cl_v,
              sc_c, x1_c, y1_c, x2_c, y2_c, ar_c,
              st_v, all_v, out_v,
              sh0, sh1, dsem):
    wid = lax.axis_index("s")
    base = wid * _C

    cps = [
        pltpu.async_copy(cx_h, cx_v.at[pl.ds(0, _N)], dsem),
        pltpu.async_copy(cy_h, cy_v.at[pl.ds(0, _N)], dsem),
        pltpu.async_copy(w_h, w_v.at[pl.ds(0, _N)], dsem),
        pltpu.async_copy(h_h, h_v.at[pl.ds(0, _N)], dsem),
        pltpu.async_copy(cl_h, cl_v.at[pl.ds(0, _N)], dsem),
    ]
    @pl.when(wid < _NW - 1)
    def _():
        pltpu.async_copy(sc_h.at[pl.ds(base, _C)], sc_c, dsem).wait()
    @pl.when(wid == _NW - 1)
    def _():
        pltpu.async_copy(
            sc_h.at[pl.ds((_NW - 1) * _C, _TAIL)],
            sc_c.at[pl.ds(0, _TAIL)], dsem).wait()
    for cp in cps:
        cp.wait()

    lane = lax.iota(jnp.int32, _L)
    neg2 = jnp.full((_L,), -2.0, jnp.float32)
    zero_i = jnp.zeros((_L,), jnp.int32)

    def conv_amax(i, carry):
        bv, bi = carry
        for u in range(2):
            j = 2 * i + u
            sl = pl.ds(j * _L, _L)
            gsl = pl.ds(base + j * _L, _L)
            gidx = lane + (base + j * _L)
            cx = cx_v[gsl]
            cy = cy_v[gsl]
            w = w_v[gsl]
            h = h_v[gsl]
            off = cl_v[gsl].astype(jnp.float32) * _OFF
            x1 = _clip(cx - w / 2.0) + off
            y1 = _clip(cy - h / 2.0) + off
            x2 = _clip(cx + w / 2.0) + off
            y2 = _clip(cy + h / 2.0) + off
            x1_c[sl] = x1
            y1_c[sl] = y1
            x2_c[sl] = x2
            y2_c[sl] = y2
            ar_c[sl] = (x2 - x1 + 1.0) * (y2 - y1 + 1.0)
            s = jnp.where(gidx < _N, sc_c[sl], -1.0)
            sc_c[sl] = s
            upd = s > bv
            bv = jnp.where(upd, s, bv)
            bi = jnp.where(upd, gidx, bi)
        return bv, bi

    bv, bi = lax.fori_loop(0, _CI, conv_amax, (neg2, zero_i))

    def nms_round(k, carry):
        bv, bi = carry

        st_v[pl.ds(0, _L)] = bv
        st_v[pl.ds(_L, _L)] = plsc.bitcast(bi, jnp.float32)
        even = k % 2 == 0

        @pl.when(even)
        def _():
            pltpu.sync_copy(st_v, sh0.at[pl.ds(wid * 2 * _L, 2 * _L)])
        @pl.when(jnp.logical_not(even))
        def _():
            pltpu.sync_copy(st_v, sh1.at[pl.ds(wid * 2 * _L, 2 * _L)])
        plsc.subcore_barrier()
        @pl.when(even)
        def _():
            pltpu.sync_copy(sh0, all_v)
        @pl.when(jnp.logical_not(even))
        def _():
            pltpu.sync_copy(sh1, all_v)

        def red(r, carry):
            v2 = all_v[pl.ds(r * 2 * _L, _L)]
            i2 = plsc.bitcast(all_v[pl.ds(r * 2 * _L + _L, _L)], jnp.int32)
            return _better(v2, i2, *carry)

        bv = all_v[pl.ds(0, _L)]
        bi = plsc.bitcast(all_v[pl.ds(_L, _L)], jnp.int32)
        bv, bi = lax.fori_loop(1, _NW, red, (bv, bi))
        for s in (1, 2, 4, 8):
            p = lane ^ s
            bv2 = bv.at[p].get(mode="promise_in_bounds")
            bi2 = bi.at[p].get(mode="promise_in_bounds")
            bv, bi = _better(bv2, bi2, bv, bi)
        mv = bv
        giv = bi

        gp = jnp.minimum(giv, _N - 1)
        pcx = plsc.load_gather(cx_v, [gp])
        pcy = plsc.load_gather(cy_v, [gp])
        pw = plsc.load_gather(w_v, [gp])
        ph = plsc.load_gather(h_v, [gp])
        pcl = plsc.load_gather(cl_v, [gp]).astype(jnp.float32)
        offp = pcl * _OFF
        ux1 = _clip(pcx - pw / 2.0)
        uy1 = _clip(pcy - ph / 2.0)
        ux2 = _clip(pcx + pw / 2.0)
        uy2 = _clip(pcy + ph / 2.0)
        px1 = ux1 + offp
        py1 = uy1 + offp
        px2 = ux2 + offp
        py2 = uy2 + offp
        pa = (px2 - px1 + 1.0) * (py2 - py1 + 1.0)

        row = jnp.where(lane == 0, ux1, 0.0)
        row = jnp.where(lane == 1, uy1, row)
        row = jnp.where(lane == 2, ux2, row)
        row = jnp.where(lane == 3, uy2, row)
        row = jnp.where(lane == 4, mv, row)
        row = jnp.where(lane == 5, pcl, row)
        out_v[pl.ds(k * _L, _L)] = row

        def sup_amax(i, carry):
            bv, bi = carry
            for u in range(2):
                j = 2 * i + u
                sl = pl.ds(j * _L, _L)
                x1 = x1_c[sl]
                y1 = y1_c[sl]
                x2 = x2_c[sl]
                y2 = y2_c[sl]
                a = ar_c[sl]
                ix1 = jnp.maximum(px1, x1)
                iy1 = jnp.maximum(py1, y1)
                ix2 = jnp.minimum(px2, x2)
                iy2 = jnp.minimum(py2, y2)
                inter = (jnp.maximum(ix2 - ix1 + 1.0, 0.0)
                         * jnp.maximum(iy2 - iy1 + 1.0, 0.0))
                iou = inter / (pa + a - inter + 1e-16)
                s = jnp.where(iou > _THRES, -1.0, sc_c[sl])
                sc_c[sl] = s
                upd = s > bv
                bv = jnp.where(upd, s, bv)
                bi = jnp.where(upd, lane + (base + j * _L), bi)
            return bv, bi

        return lax.fori_loop(0, _CI, sup_amax, (neg2, zero_i))

    lax.fori_loop(0, _K, nms_round, (bv, bi))

    @pl.when(wid == 0)
    def _():
        pltpu.sync_copy(out_v, out_h)


_nms_sc = functools.partial(
    pl.kernel,
    out_type=jax.ShapeDtypeStruct((_K * _L,), jnp.float32),
    mesh=plsc.VectorSubcoreMesh(
        core_axis_name="c", subcore_axis_name="s", num_cores=1),
    compiler_params=pltpu.CompilerParams(
        needs_layout_passes=False, use_tc_tiling_on_sc=False),
    scratch_types=[
        pltpu.VMEM((_NPAD,), jnp.float32),
        pltpu.VMEM((_NPAD,), jnp.float32),
        pltpu.VMEM((_NPAD,), jnp.float32),
        pltpu.VMEM((_NPAD,), jnp.float32),
        pltpu.VMEM((_NPAD,), jnp.int32),
        pltpu.VMEM((_C,), jnp.float32),
        pltpu.VMEM((_C,), jnp.float32),
        pltpu.VMEM((_C,), jnp.float32),
        pltpu.VMEM((_C,), jnp.float32),
        pltpu.VMEM((_C,), jnp.float32),
        pltpu.VMEM((_C,), jnp.float32),
        pltpu.VMEM((2 * _L,), jnp.float32),
        pltpu.VMEM((_NW * 2 * _L,), jnp.float32),
        pltpu.VMEM((_K * _L,), jnp.float32),
        pltpu.VMEM_SHARED((_NW * 2 * _L,), jnp.float32),
        pltpu.VMEM_SHARED((_NW * 2 * _L,), jnp.float32),
        pltpu.SemaphoreType.DMA,
    ],
)(_nms_body)


def kernel(boxes, scores, idxs):
    out = _nms_sc(boxes[:, 0], boxes[:, 1], boxes[:, 2], boxes[:, 3],
                  scores, idxs)
    return out.reshape(_K, _L)[:, :6]
